# baseline (speedup 1.0000x reference)
import functools

import jax
import jax.numpy as jnp
from jax import lax
from jax.experimental import pallas as pl
from jax.experimental.pallas import tpu as pltpu
from jax.experimental.pallas import tpu_sc as plsc

_NC = 2
_NS = 16
_L = 16
_BLK = 128
_CBLK = 125
_CHUNK = _CBLK * _BLK


def _make_sc_segsum(n_nodes, n_edges, d_edge):
    epc = n_edges // _NC
    nch = epc // _CHUNK
    hrows = n_nodes // _L
    crows = -(-hrows // _NS) * _NS
    czrows = crows // _NS
    mrg = -(-hrows // 125)

    mesh = plsc.VectorSubcoreMesh(core_axis_name="c", subcore_axis_name="s")

    @functools.partial(
        pl.kernel,
        out_type=(
            jax.ShapeDtypeStruct((_NC, d_edge, n_nodes // _L, _L), jnp.float32),
            jax.ShapeDtypeStruct((_NC, crows, _L), jnp.float32),
        ),
        mesh=mesh,
        compiler_params=pltpu.CompilerParams(use_tc_tiling_on_sc=False,
                                             needs_layout_passes=False),
        scratch_types=[
            pltpu.VMEM((2, _CBLK, _BLK), jnp.float32),
            pltpu.VMEM((2, _CBLK, _BLK), jnp.int32),
            pltpu.VMEM((mrg, 125), jnp.int32),
            pltpu.VMEM((n_nodes // _L, _L), jnp.float32),
            pltpu.VMEM((n_nodes // _L, _L), jnp.float32),
            pltpu.VMEM((hrows, _L), jnp.float32),
            pltpu.VMEM_SHARED((crows, _L), jnp.float32),
            pltpu.SemaphoreType.DMA,
            pltpu.SemaphoreType.DMA,
        ],
    )
    def sc_segsum(feat_hbm, col_hbm, midx_hbm, sums_out, cnt_out,
                  val_v, colc_v, midx_v, sum_v, sumb_v, hist_v, cnt_sh,
                  gsem, msem):
        cid = lax.axis_index("c")
        sid = lax.axis_index("s")
        fb = sid // 8
        fr = sid % 8

        def zfill(ref, n):
            def body(i, _):
                ref[i, :] = jnp.zeros((_L,), jnp.float32)
                return 0
            lax.fori_loop(0, n, body, 0)

        zfill(sum_v, n_nodes // _L)
        zfill(sumb_v, n_nodes // _L)
        zfill(hist_v, hrows)
        pltpu.sync_copy(hist_v.at[pl.ds(0, czrows)],
                        cnt_sh.at[pl.ds(sid * czrows, czrows)])
        pltpu.sync_copy(midx_hbm, midx_v)
        plsc.subcore_barrier()

        ebase = cid * (epc // _BLK)

        def gather(k, b):
            d0 = pltpu.async_copy(
                feat_hbm.at[fb, pl.ds(ebase + k * _CBLK, _CBLK), fr],
                val_v.at[b], gsem)
            d1 = pltpu.async_copy(
                col_hbm.at[cid, pl.ds(k * _CBLK, _CBLK)], colc_v.at[b], gsem)
            return (d0, d1)

        ones = jnp.ones((_L,), jnp.float32)
        clo = (sid * _CBLK) // _NS
        chi = ((sid + 1) * _CBLK) // _NS

        def process(b):
            def body(i, _):
                for l in range(_BLK // _L):
                    c = colc_v[b, i, pl.ds(l * _L, _L)]
                    v = val_v[b, i, pl.ds(l * _L, _L)]
                    tgt = sum_v if l % 2 == 0 else sumb_v
                    plsc.addupdate_scatter(tgt, [c >> 4, c & 15], v)
                return 0
            lax.fori_loop(0, _CBLK, body, 0)

            def cbody(i, _):
                for l in range(_BLK // _L):
                    c = colc_v[b, i, pl.ds(l * _L, _L)]
                    plsc.addupdate_scatter(hist_v, [c >> 4, c & 15], ones)
                return 0
            lax.fori_loop(clo, chi, cbody, 0)

        desc = [None, None]
        desc[0] = gather(0, 0)
        for k in range(nch):
            b = k % 2
            desc[b][0].wait()
            desc[b][1].wait()
            if k + 1 < nch:
                desc[1 - b] = gather(k + 1, 1 - b)
            process(b)

        def combine(i, _):
            sum_v[i, :] = sum_v[i, :] + sumb_v[i, :]
            return 0
        lax.fori_loop(0, n_nodes // _L, combine, 0)
        pltpu.sync_copy(sum_v, sums_out.at[cid, sid])

        def mbody(c, _):
            pltpu.async_copy(hist_v.at[pl.ds(c * 125, 125)],
                             cnt_sh.at[midx_v.at[c]], msem, add=True)
            return 0
        lax.fori_loop(0, mrg, mbody, 0)
        pltpu.make_async_copy(cnt_out.at[cid, pl.ds(0, hrows)],
                              hist_v, msem).wait()
        plsc.subcore_barrier()

        @pl.when(sid == 0)
        def _():
            pltpu.sync_copy(cnt_sh, cnt_out.at[cid])

    return sc_segsum


def _tc_dense(x, s0, s1, c0, c1, batch2d, u, w1x, w1e, w1u, b1, gamma, beta,
              w2, b2):
    n, d_node = x.shape
    n_graphs = u.shape[0]
    d_out = w2.shape[1]

    def body(x_ref, s0_ref, s1_ref, c0_ref, c1_ref, b_ref, u_ref,
             w1x_ref, w1e_ref, w1u_ref, b1_ref, g_ref, be_ref, w2_ref, b2_ref,
             out_ref):
        sums_t = s0_ref[...] + s1_ref[...]
        cnt = jnp.maximum(c0_ref[...] + c1_ref[...], 1.0)
        e_t = sums_t / cnt
        he = lax.dot_general(e_t, w1e_ref[...], (((0,), (0,)), ((), ())),
                             preferred_element_type=jnp.float32)
        oh = (b_ref[...] == lax.broadcasted_iota(jnp.int32, (n, n_graphs), 1))
        ub = jnp.dot(oh.astype(jnp.float32), u_ref[...],
                     preferred_element_type=jnp.float32)
        h = (jnp.dot(x_ref[...], w1x_ref[...], preferred_element_type=jnp.float32)
             + he
             + jnp.dot(ub, w1u_ref[...], preferred_element_type=jnp.float32)
             + b1_ref[...])
        mean = jnp.mean(h, axis=0, keepdims=True)
        var = jnp.mean((h - mean) ** 2, axis=0, keepdims=True)
        hn = (h - mean) / jnp.sqrt(var + 1e-5) * g_ref[...] + be_ref[...]
        hr = jnp.maximum(hn, 0.0)
        out_ref[...] = (jnp.dot(hr, w2_ref[...], preferred_element_type=jnp.float32)
                        + b2_ref[...])

    return pl.pallas_call(
        body,
        out_shape=jax.ShapeDtypeStruct((n, d_out), jnp.float32),
    )(x, s0, s1, c0, c1, batch2d, u, w1x, w1e, w1u, b1, gamma, beta, w2, b2)


def kernel(x, edge_index, edge_attr, u, batch, W1, b1, gamma, beta, W2, b2):
    n_nodes, d_node = x.shape
    n_edges, d_edge = edge_attr.shape
    hidden = W1.shape[0]

    col = edge_index[1].astype(jnp.int32)
    colc = col.reshape(_NC, n_edges // _NC // _BLK, _BLK)
    featv = edge_attr.T.reshape(d_edge // 8, 8, n_edges // _BLK, _BLK)
    featv = featv.transpose(0, 2, 1, 3)
    hrows = n_nodes // _L
    mrg = -(-hrows // 125)
    midx = jnp.arange(mrg * 125, dtype=jnp.int32).reshape(mrg, 125)

    sc_segsum = _make_sc_segsum(n_nodes, n_edges, d_edge)
    sums, cnt = sc_segsum(featv, colc, midx)
    sums = sums.reshape(_NC, d_edge, n_nodes)

    crows = cnt.shape[1]
    c0 = cnt[0].reshape(1, crows * _L)[:, :n_nodes]
    c1 = cnt[1].reshape(1, crows * _L)[:, :n_nodes]

    w1t = W1.T
    w1x = w1t[:d_node]
    w1e = w1t[d_node:d_node + d_edge]
    w1u = w1t[d_node + d_edge:]

    return _tc_dense(
        x, sums[0], sums[1], c0, c1,
        batch.astype(jnp.int32).reshape(n_nodes, 1), u,
        w1x, w1e, w1u,
        b1.reshape(1, hidden), gamma.reshape(1, hidden), beta.reshape(1, hidden),
        W2.T, b2.reshape(1, W2.shape[0]),
    )


# speedup vs baseline: 1.1436x; 1.1436x over previous
import functools

import jax
import jax.numpy as jnp
from jax import lax
from jax.experimental import pallas as pl
from jax.experimental.pallas import tpu as pltpu
from jax.experimental.pallas import tpu_sc as plsc

_NC = 2
_NS = 16
_L = 16
_BLK = 128
_CBLK = 125
_CHUNK = _CBLK * _BLK


def _make_sc_segsum(n_nodes, n_edges, d_edge):
    epc = n_edges // _NC
    nch = epc // _CHUNK
    hrows = n_nodes // _L
    crows = -(-hrows // _NS) * _NS
    czrows = crows // _NS
    mrg = -(-hrows // 125)

    mesh = plsc.VectorSubcoreMesh(core_axis_name="c", subcore_axis_name="s")

    @functools.partial(
        pl.kernel,
        out_type=(
            jax.ShapeDtypeStruct((_NC, d_edge, n_nodes // _L, _L), jnp.float32),
            jax.ShapeDtypeStruct((_NC, crows, _L), jnp.float32),
        ),
        mesh=mesh,
        compiler_params=pltpu.CompilerParams(use_tc_tiling_on_sc=False,
                                             needs_layout_passes=False),
        scratch_types=[
            pltpu.VMEM((2, _CBLK, _BLK), jnp.float32),
            pltpu.VMEM((2, _CBLK, _BLK), jnp.int32),
            pltpu.VMEM((mrg, 125), jnp.int32),
            pltpu.VMEM((n_nodes // _L, _L), jnp.float32),
            pltpu.VMEM((hrows, _L), jnp.float32),
            pltpu.VMEM_SHARED((crows, _L), jnp.float32),
            pltpu.SemaphoreType.DMA,
            pltpu.SemaphoreType.DMA,
        ],
    )
    def sc_segsum(feat_hbm, col_hbm, midx_hbm, sums_out, cnt_out,
                  val_v, colc_v, midx_v, sum_v, hist_v, cnt_sh, gsem, msem):
        cid = lax.axis_index("c")
        sid = lax.axis_index("s")
        fb = sid // 8
        fr = sid % 8

        def zfill(ref, n):
            def body(i, _):
                ref[i, :] = jnp.zeros((_L,), jnp.float32)
                return 0
            lax.fori_loop(0, n, body, 0)

        zfill(sum_v, n_nodes // _L)
        zfill(hist_v, hrows)
        pltpu.sync_copy(hist_v.at[pl.ds(0, czrows)],
                        cnt_sh.at[pl.ds(sid * czrows, czrows)])
        pltpu.sync_copy(midx_hbm, midx_v)
        plsc.subcore_barrier()

        ebase = cid * (epc // _BLK)

        def gather(k, b):
            d0 = pltpu.async_copy(
                feat_hbm.at[fb, pl.ds(ebase + k * _CBLK, _CBLK), fr],
                val_v.at[b], gsem)
            d1 = pltpu.async_copy(
                col_hbm.at[pl.ds(ebase + k * _CBLK, _CBLK), 1],
                colc_v.at[b], gsem)
            return (d0, d1)

        ones = jnp.ones((_L,), jnp.float32)
        clo = (sid * _CBLK) // _NS
        chi = ((sid + 1) * _CBLK) // _NS

        def process(b):
            def body(i, _):
                for l in range(_BLK // _L):
                    c = colc_v[b, i, pl.ds(l * _L, _L)]
                    v = val_v[b, i, pl.ds(l * _L, _L)]
                    plsc.addupdate_scatter(sum_v, [c >> 4, c & 15], v)
                return 0
            lax.fori_loop(0, _CBLK, body, 0)

            def cbody(i, _):
                for l in range(_BLK // _L):
                    c = colc_v[b, i, pl.ds(l * _L, _L)]
                    plsc.addupdate_scatter(hist_v, [c >> 4, c & 15], ones)
                return 0
            lax.fori_loop(clo, chi, cbody, 0)

        desc = [None, None]
        desc[0] = gather(0, 0)
        for k in range(nch):
            b = k % 2
            desc[b][0].wait()
            desc[b][1].wait()
            if k + 1 < nch:
                desc[1 - b] = gather(k + 1, 1 - b)
            process(b)

        pltpu.sync_copy(sum_v, sums_out.at[cid, sid])

        def mbody(c, _):
            pltpu.async_copy(hist_v.at[pl.ds(c * 125, 125)],
                             cnt_sh.at[midx_v.at[c]], msem, add=True)
            return 0
        lax.fori_loop(0, mrg, mbody, 0)
        pltpu.make_async_copy(cnt_out.at[cid, pl.ds(0, hrows)],
                              hist_v, msem).wait()
        plsc.subcore_barrier()

        @pl.when(sid == 0)
        def _():
            pltpu.sync_copy(cnt_sh, cnt_out.at[cid])

    return sc_segsum


def _tc_dense(x, s0, s1, c0, c1, batch2d, u, w1, b1, gamma, beta, w2, b2):
    n, d_node = x.shape
    n_graphs, d_graph = u.shape
    d_edge = s0.shape[0]
    d_out = w2.shape[0]

    def body(x_ref, s0_ref, s1_ref, c0_ref, c1_ref, b_ref, u_ref,
             w1_ref, b1_ref, g_ref, be_ref, w2_ref, b2_ref, out_ref):
        sums_t = s0_ref[...] + s1_ref[...]
        cnt = jnp.maximum(c0_ref[...] + c1_ref[...], 1.0)
        e_t = sums_t / cnt
        he = lax.dot_general(e_t, w1_ref[:, d_node:d_node + d_edge],
                             (((0,), (1,)), ((), ())),
                             preferred_element_type=jnp.float32)
        oh = (b_ref[...] == lax.broadcasted_iota(jnp.int32, (n, n_graphs), 1))
        ub = jnp.dot(oh.astype(jnp.float32), u_ref[...],
                     preferred_element_type=jnp.float32)
        h = (lax.dot_general(x_ref[...], w1_ref[:, :d_node],
                             (((1,), (1,)), ((), ())),
                             preferred_element_type=jnp.float32)
             + he
             + lax.dot_general(ub, w1_ref[:, d_node + d_edge:],
                               (((1,), (1,)), ((), ())),
                               preferred_element_type=jnp.float32)
             + b1_ref[...])
        mean = jnp.mean(h, axis=0, keepdims=True)
        var = jnp.mean((h - mean) ** 2, axis=0, keepdims=True)
        hn = (h - mean) / jnp.sqrt(var + 1e-5) * g_ref[...] + be_ref[...]
        hr = jnp.maximum(hn, 0.0)
        out_ref[...] = (lax.dot_general(hr, w2_ref[...],
                                        (((1,), (1,)), ((), ())),
                                        preferred_element_type=jnp.float32)
                        + b2_ref[...])

    return pl.pallas_call(
        body,
        out_shape=jax.ShapeDtypeStruct((n, d_out), jnp.float32),
    )(x, s0, s1, c0, c1, batch2d, u, w1, b1, gamma, beta, w2, b2)


def kernel(x, edge_index, edge_attr, u, batch, W1, b1, gamma, beta, W2, b2):
    n_nodes, d_node = x.shape
    n_edges, d_edge = edge_attr.shape
    hidden = W1.shape[0]

    colc = edge_index.astype(jnp.int32).reshape(2, n_edges // _BLK, _BLK)
    colc = colc.transpose(1, 0, 2)
    featv = edge_attr.T.reshape(d_edge // 8, 8, n_edges // _BLK, _BLK)
    featv = featv.transpose(0, 2, 1, 3)
    hrows = n_nodes // _L
    mrg = -(-hrows // 125)
    midx = jnp.arange(mrg * 125, dtype=jnp.int32).reshape(mrg, 125)

    sc_segsum = _make_sc_segsum(n_nodes, n_edges, d_edge)
    sums, cnt = sc_segsum(featv, colc, midx)
    sums = sums.reshape(_NC, d_edge, n_nodes)

    crows = cnt.shape[1]
    c0 = cnt[0].reshape(1, crows * _L)[:, :n_nodes]
    c1 = cnt[1].reshape(1, crows * _L)[:, :n_nodes]

    return _tc_dense(
        x, sums[0], sums[1], c0, c1,
        batch.astype(jnp.int32).reshape(n_nodes, 1), u, W1,
        b1.reshape(1, hidden), gamma.reshape(1, hidden), beta.reshape(1, hidden),
        W2, b2.reshape(1, W2.shape[0]),
    )


# speedup vs baseline: 1.1683x; 1.0216x over previous
import functools

import jax
import jax.numpy as jnp
from jax import lax
from jax.experimental import pallas as pl
from jax.experimental.pallas import tpu as pltpu
from jax.experimental.pallas import tpu_sc as plsc

_NC = 2
_NS = 16
_L = 16
_BLK = 128
_CBLK = 125
_CHUNK = _CBLK * _BLK


def _make_sc_segsum(n_nodes, n_edges, d_edge):
    epc = n_edges // _NC
    nch = epc // _CHUNK
    hrows = n_nodes // _L
    crows = -(-hrows // _NS) * _NS
    czrows = crows // _NS
    mrg = -(-hrows // 125)

    mesh = plsc.VectorSubcoreMesh(core_axis_name="c", subcore_axis_name="s")

    @functools.partial(
        pl.kernel,
        out_type=(
            jax.ShapeDtypeStruct((_NC, d_edge, n_nodes // _L, _L), jnp.float32),
            jax.ShapeDtypeStruct((_NC, crows, _L), jnp.float32),
        ),
        mesh=mesh,
        compiler_params=pltpu.CompilerParams(use_tc_tiling_on_sc=False,
                                             needs_layout_passes=False),
        scratch_types=[
            pltpu.VMEM((2, _CBLK, _BLK), jnp.float32),
            pltpu.VMEM((2, _CBLK, _BLK), jnp.int32),
            pltpu.VMEM((mrg, 125), jnp.int32),
            pltpu.VMEM((n_nodes // _L, _L), jnp.float32),
            pltpu.VMEM((hrows, _L), jnp.float32),
            pltpu.VMEM_SHARED((crows, _L), jnp.float32),
            pltpu.SemaphoreType.DMA,
            pltpu.SemaphoreType.DMA,
        ],
    )
    def sc_segsum(feat_hbm, col_hbm, midx_hbm, sums_out, cnt_out,
                  val_v, colc_v, midx_v, sum_v, hist_v, cnt_sh, gsem, msem):
        cid = lax.axis_index("c")
        sid = lax.axis_index("s")
        fb = sid // 8
        fr = sid % 8

        def zfill(ref, n):
            def body(i, _):
                ref[i, :] = jnp.zeros((_L,), jnp.float32)
                return 0
            lax.fori_loop(0, n, body, 0)

        zfill(sum_v, n_nodes // _L)
        zfill(hist_v, hrows)
        pltpu.sync_copy(hist_v.at[pl.ds(0, czrows)],
                        cnt_sh.at[pl.ds(sid * czrows, czrows)])
        pltpu.sync_copy(midx_hbm, midx_v)
        plsc.subcore_barrier()

        ebase = cid * (epc // _BLK)

        def gather(k, b):
            d0 = pltpu.async_copy(
                feat_hbm.at[fb, pl.ds(ebase + k * _CBLK, _CBLK), fr],
                val_v.at[b], gsem)
            d1 = pltpu.async_copy(
                col_hbm.at[pl.ds(ebase + k * _CBLK, _CBLK), 1],
                colc_v.at[b], gsem)
            return (d0, d1)

        ones = jnp.ones((_L,), jnp.float32)
        clo = (sid * _CBLK) // _NS
        chi = ((sid + 1) * _CBLK) // _NS

        def process(b):
            def body(i, _):
                for l in range(_BLK // _L):
                    c = colc_v[b, i, pl.ds(l * _L, _L)]
                    v = val_v[b, i, pl.ds(l * _L, _L)]
                    plsc.addupdate_scatter(sum_v, [c >> 4, c & 15], v)
                return 0
            lax.fori_loop(0, _CBLK, body, 0)

            def cbody(i, _):
                for l in range(_BLK // _L):
                    c = colc_v[b, i, pl.ds(l * _L, _L)]
                    plsc.addupdate_scatter(hist_v, [c >> 4, c & 15], ones)
                return 0
            lax.fori_loop(clo, chi, cbody, 0)

        desc = [None, None]
        desc[0] = gather(0, 0)
        for k in range(nch):
            b = k % 2
            desc[b][0].wait()
            desc[b][1].wait()
            if k + 1 < nch:
                desc[1 - b] = gather(k + 1, 1 - b)
            process(b)

        pltpu.sync_copy(sum_v, sums_out.at[cid, sid])

        def mbody(c, _):
            pltpu.async_copy(hist_v.at[pl.ds(c * 125, 125)],
                             cnt_sh.at[midx_v.at[c]], msem, add=True)
            return 0
        lax.fori_loop(0, mrg, mbody, 0)
        pltpu.make_async_copy(cnt_out.at[cid, pl.ds(0, hrows)],
                              hist_v, msem).wait()
        plsc.subcore_barrier()

        @pl.when(sid == 0)
        def _():
            pltpu.sync_copy(cnt_sh, cnt_out.at[cid])

    return sc_segsum


def _tc_pre(x, batch2d, u, w1, b1):
    n, d_node = x.shape
    n_graphs, d_graph = u.shape
    hidden = w1.shape[0]
    d_edge = w1.shape[1] - d_node - d_graph

    def body(x_ref, b_ref, u_ref, w1_ref, b1_ref, out_ref):
        oh = (b_ref[...] == lax.broadcasted_iota(jnp.int32, (n, n_graphs), 1))
        ub = jnp.dot(oh.astype(jnp.float32), u_ref[...],
                     preferred_element_type=jnp.float32)
        out_ref[...] = (lax.dot_general(x_ref[...], w1_ref[:, :d_node],
                                        (((1,), (1,)), ((), ())),
                                        preferred_element_type=jnp.float32)
                        + lax.dot_general(ub, w1_ref[:, d_node + d_edge:],
                                          (((1,), (1,)), ((), ())),
                                          preferred_element_type=jnp.float32)
                        + b1_ref[...])

    return pl.pallas_call(
        body,
        out_shape=jax.ShapeDtypeStruct((n, hidden), jnp.float32),
    )(x, batch2d, u, w1, b1)


def _tc_dense(hpre, s0, s1, c0, c1, d_node, w1, gamma, beta, w2, b2):
    n, hidden = hpre.shape
    d_edge = s0.shape[0]
    d_out = w2.shape[0]

    def body(hp_ref, s0_ref, s1_ref, c0_ref, c1_ref,
             w1_ref, g_ref, be_ref, w2_ref, b2_ref, out_ref):
        sums_t = s0_ref[...] + s1_ref[...]
        cnt = jnp.maximum(c0_ref[...] + c1_ref[...], 1.0)
        e_t = sums_t / cnt
        he = lax.dot_general(e_t, w1_ref[:, d_node:d_node + d_edge],
                             (((0,), (1,)), ((), ())),
                             preferred_element_type=jnp.float32)
        h = hp_ref[...] + he
        mean = jnp.mean(h, axis=0, keepdims=True)
        var = jnp.mean((h - mean) ** 2, axis=0, keepdims=True)
        hn = (h - mean) / jnp.sqrt(var + 1e-5) * g_ref[...] + be_ref[...]
        hr = jnp.maximum(hn, 0.0)
        out_ref[...] = (lax.dot_general(hr, w2_ref[...],
                                        (((1,), (1,)), ((), ())),
                                        preferred_element_type=jnp.float32)
                        + b2_ref[...])

    return pl.pallas_call(
        body,
        out_shape=jax.ShapeDtypeStruct((n, d_out), jnp.float32),
    )(hpre, s0, s1, c0, c1, w1, gamma, beta, w2, b2)


def kernel(x, edge_index, edge_attr, u, batch, W1, b1, gamma, beta, W2, b2):
    n_nodes, d_node = x.shape
    n_edges, d_edge = edge_attr.shape
    hidden = W1.shape[0]

    colc = edge_index.astype(jnp.int32).reshape(2, n_edges // _BLK, _BLK)
    colc = colc.transpose(1, 0, 2)
    featv = edge_attr.T.reshape(d_edge // 8, 8, n_edges // _BLK, _BLK)
    featv = featv.transpose(0, 2, 1, 3)
    hrows = n_nodes // _L
    mrg = -(-hrows // 125)
    midx = jnp.arange(mrg * 125, dtype=jnp.int32).reshape(mrg, 125)

    sc_segsum = _make_sc_segsum(n_nodes, n_edges, d_edge)
    sums, cnt = sc_segsum(featv, colc, midx)
    sums = sums.reshape(_NC, d_edge, n_nodes)

    crows = cnt.shape[1]
    c0 = cnt[0].reshape(1, crows * _L)[:, :n_nodes]
    c1 = cnt[1].reshape(1, crows * _L)[:, :n_nodes]

    hpre = _tc_pre(x, batch.astype(jnp.int32).reshape(n_nodes, 1), u, W1,
                   b1.reshape(1, hidden))
    return _tc_dense(
        hpre, sums[0], sums[1], c0, c1, d_node, W1,
        gamma.reshape(1, hidden), beta.reshape(1, hidden),
        W2, b2.reshape(1, W2.shape[0]),
    )
